# Initial kernel scaffold; baseline (speedup 1.0000x reference)
import jax
import jax.numpy as jnp
from jax.experimental import pallas as pl


def kernel(target, cluster_centers):
    raise NotImplementedError("write your pallas kernel here")


# speedup vs baseline: 13.9273x; 13.9273x over previous
import functools

import jax
import jax.numpy as jnp
from jax.experimental import pallas as pl
from jax.experimental.pallas import tpu as pltpu


def _kmeans_body(x_ref, c_ref, fill_ref, means_ref, covs_ref, counts_ref,
                 pred_ref, nsteps):
    i = pl.program_id(0)
    x = x_ref[...]
    c = c_ref[...]
    bn = x.shape[0]
    k = c.shape[0]

    xx = jnp.sum(x * x, axis=1, keepdims=True)
    cc = jnp.sum(c * c, axis=1)[None, :]
    xc = jax.lax.dot_general(
        x, c, (((1,), (1,)), ((), ())),
        preferred_element_type=jnp.float32)
    d = xx + cc - 2.0 * xc

    pred = jnp.argmin(d, axis=1).astype(jnp.int32)
    pred_ref[...] = pred

    rowmin = jnp.min(d, axis=1, keepdims=True)
    e = jnp.exp(rowmin - d)
    soft = e / jnp.sum(e, axis=1, keepdims=True)
    fill_c = jnp.sum(soft, axis=0)

    iota = jax.lax.broadcasted_iota(jnp.int32, (bn, k), 1)
    p = (iota == pred[:, None]).astype(jnp.float32)

    counts_c = jnp.sum(p, axis=0)
    sx_c = jax.lax.dot_general(
        p, x, (((0,), (0,)), ((), ())),
        preferred_element_type=jnp.float32)
    of = (x[:, :, None] * x[:, None, :]).reshape(bn, 32 * 32)
    s2_c = jax.lax.dot_general(
        p, of, (((0,), (0,)), ((), ())),
        preferred_element_type=jnp.float32)

    @pl.when(i == 0)
    def _init():
        fill_ref[...] = fill_c
        counts_ref[...] = counts_c
        means_ref[...] = sx_c
        covs_ref[...] = s2_c

    @pl.when(i > 0)
    def _accum():
        fill_ref[...] += fill_c
        counts_ref[...] += counts_c
        means_ref[...] += sx_c
        covs_ref[...] += s2_c

    @pl.when(i == nsteps - 1)
    def _finalize():
        counts = counts_ref[...]
        safe = jnp.maximum(counts, 1.0)
        inv = 1.0 / safe
        means = means_ref[...] * inv[:, None]
        means_ref[...] = means
        mm = (means[:, :, None] * means[:, None, :]).reshape(k, 32 * 32)
        covs_ref[...] = (covs_ref[...] - counts[:, None] * mm) * inv[:, None]
        fill_ref[...] = fill_ref[...] * (1.0 / (nsteps * bn))


@functools.partial(jax.jit, static_argnames=())
def kernel(target, cluster_centers):
    n, dim = target.shape
    k = cluster_centers.shape[0]
    bn = 2048
    nsteps = n // bn

    fill, means, covs_flat, _counts, pred = pl.pallas_call(
        functools.partial(_kmeans_body, nsteps=nsteps),
        grid=(nsteps,),
        in_specs=[
            pl.BlockSpec((bn, dim), lambda i: (i, 0)),
            pl.BlockSpec((k, dim), lambda i: (0, 0)),
        ],
        out_specs=[
            pl.BlockSpec((k,), lambda i: (0,)),
            pl.BlockSpec((k, dim), lambda i: (0, 0)),
            pl.BlockSpec((k, dim * dim), lambda i: (0, 0)),
            pl.BlockSpec((k,), lambda i: (0,)),
            pl.BlockSpec((bn,), lambda i: (i,)),
        ],
        out_shape=[
            jax.ShapeDtypeStruct((k,), jnp.float32),
            jax.ShapeDtypeStruct((k, dim), jnp.float32),
            jax.ShapeDtypeStruct((k, dim * dim), jnp.float32),
            jax.ShapeDtypeStruct((k,), jnp.float32),
            jax.ShapeDtypeStruct((n,), jnp.int32),
        ],
    )(target, cluster_centers)

    return fill, means, covs_flat.reshape(k, dim, dim), pred
